# baseline (device time: 29041 ns/iter reference)
import jax
import jax.numpy as jnp
from jax import lax
from jax.experimental import pallas as pl
from jax.experimental.pallas import tpu as pltpu

N_DEV = 4
N_STREAMS = 4
ORDER = (0, 2, 1, 3)


def kernel(x, W1, W2):
    m, _ = x.shape
    d = W1.shape[1]
    n = W2.shape[1]
    mc = m // N_DEV
    qh = mc // N_STREAMS
    bf16 = jnp.bfloat16
    f32 = jnp.float32

    def body(x_ref, w1_ref, w2_ref, out_ref,
             h_ref, comm_ref, ag_ref, ostage_ref,
             out_sems, rs_send, rs_recv, ag_send, ag_recv):
        w1 = w1_ref[...]
        w2 = w2_ref[...]
        p = lax.axis_index("i")
        left = lax.rem(p + N_DEV - 1, N_DEV)
        right = lax.rem(p + 1, N_DEV)

        def mod4(v):
            return lax.rem(v + 4 * N_DEV, N_DEV)

        def is_r(st):
            return st < 2

        def row_start(c, st):
            return c * mc + st * qh

        def h_q(c, st):
            return h_ref[pl.ds(row_start(c, st), qh), :]

        def nbr(st):
            return right if is_r(st) else left

        def rs_id(st, s):
            return mod4(p - s - 1) if is_r(st) else mod4(p + s + 1)

        def ag_id(st, t):
            return mod4(p - t) if is_r(st) else mod4(p + t)

        def make(src_ref, buf, st, slot_dst, send_sems, recv_sems, hop):
            return pltpu.make_async_remote_copy(
                src_ref=src_ref,
                dst_ref=buf.at[st, slot_dst],
                send_sem=send_sems.at[st, hop],
                recv_sem=recv_sems.at[st, hop],
                device_id=(nbr(st),),
                device_id_type=pl.DeviceIdType.MESH,
            )

        sh = qh // 2

        def make_rs(src_ref, st, slot_dst, hop, sub):
            return pltpu.make_async_remote_copy(
                src_ref=src_ref,
                dst_ref=comm_ref.at[st, slot_dst, pl.ds(sub * sh, sh)],
                send_sem=rs_send.at[st, hop, sub],
                recv_sem=rs_recv.at[st, hop, sub],
                device_id=(nbr(st),),
                device_id_type=pl.DeviceIdType.MESH,
            )

        barrier = pltpu.get_barrier_semaphore()
        for b in (left, right):
            pl.semaphore_signal(barrier, inc=1, device_id=(b,),
                                device_id_type=pl.DeviceIdType.MESH)

        def gemm1_q(st):
            start = row_start(p, st)
            h_ref[pl.ds(start, qh), :] = jnp.dot(
                x_ref[pl.ds(start, qh), :], w1,
                preferred_element_type=f32).astype(bf16)

        def gemm1(c):
            start = c * mc
            h_ref[pl.ds(start, mc), :] = jnp.dot(
                x_ref[pl.ds(start, mc), :], w1,
                preferred_element_type=f32).astype(bf16)

        rs_desc = [[[None, None] for _ in range(N_DEV - 1)]
                   for _ in range(N_STREAMS)]
        ag_desc = [[None] * (N_DEV - 1) for _ in range(N_STREAMS)]

        def start_rs0(st):
            for sub in (0, 1):
                rs_desc[st][0][sub] = make_rs(
                    h_ref.at[pl.ds(row_start(p, st) + sub * sh, sh)],
                    st, 0, 0, sub)
                rs_desc[st][0][sub].start()

        gemm1_q(0)
        gemm1_q(2)
        pl.semaphore_wait(barrier, 2)
        start_rs0(0)
        start_rs0(2)
        gemm1_q(1)
        gemm1_q(3)
        start_rs0(1)
        start_rs0(3)

        gemm1(mod4(p + 3))
        gemm1(mod4(p + 1))

        RS_EVENTS = ((0, 0), (2, 0), (0, 1), (2, 1),
                     (1, 0), (3, 0), (1, 1), (3, 1))
        for s in range(N_DEV - 1):
            for st, sub in RS_EVENTS:
                rs_desc[st][s][sub].wait_recv()
                rows = pl.ds(sub * sh, sh)
                c = rs_id(st, s)
                acc = (comm_ref[st, s, rows, :]
                       + h_ref[pl.ds(row_start(c, st) + sub * sh, sh), :])
                if s < N_DEV - 2:
                    comm_ref[st, s, rows, :] = acc
                    rs_desc[st][s + 1][sub] = make_rs(
                        comm_ref.at[st, s, rows], st, s + 1, s + 1, sub)
                    rs_desc[st][s + 1][sub].start()
                else:
                    ag_ref[st, 3, rows, :] = acc
                    if sub == 1:
                        ag_desc[st][0] = make(
                            ag_ref.at[st, 3], ag_ref, st, 0,
                            ag_send, ag_recv, 0)
                        ag_desc[st][0].start()
            if s == 0:
                gemm1(mod4(p + 2))

        out_copies = []

        def gemm2(src_val, c, st):
            ostage_ref[pl.ds(row_start(c, st), qh), :] = jnp.dot(
                src_val, w2, preferred_element_type=f32
            ).astype(bf16)

        def flush_chunk(c):
            rows = pl.ds(c * mc, mc)
            cp = pltpu.make_async_copy(
                ostage_ref.at[rows], out_ref.at[rows],
                out_sems.at[len(out_copies)])
            cp.start()
            out_copies.append(cp)

        for st in ORDER:
            own_c = mod4(p + 1) if is_r(st) else mod4(p - 1)
            gemm2(ag_ref[st, 3], own_c, st)

        for t in range(N_DEV - 1):
            if t < N_DEV - 2:
                for st in ORDER:
                    ag_desc[st][t].wait_recv()
                    ag_desc[st][t + 1] = make(
                        ag_ref.at[st, t], ag_ref, st, t + 1,
                        ag_send, ag_recv, t + 1)
                    ag_desc[st][t + 1].start()
                for st in ORDER:
                    gemm2(ag_ref[st, t], ag_id(st, t), st)
                if t == 0:
                    flush_chunk(p)
                else:
                    flush_chunk(mod4(p + 1))
                    flush_chunk(mod4(p - 1))
            else:
                for st in ORDER:
                    ag_desc[st][t].wait_recv()
                    gemm2(ag_ref[st, t], ag_id(st, t), st)
                flush_chunk(mod4(p + 2))

        for cp in out_copies:
            cp.wait()
        for st in range(N_STREAMS):
            for s in range(N_DEV - 1):
                rs_desc[st][s][0].wait_send()
                rs_desc[st][s][1].wait_send()
                ag_desc[st][s].wait_send()

    call = pl.pallas_call(
        body,
        out_shape=jax.ShapeDtypeStruct((m, n), bf16),
        in_specs=[pl.BlockSpec(memory_space=pltpu.VMEM)] * 3,
        out_specs=pl.BlockSpec(memory_space=pl.ANY),
        scratch_shapes=[
            pltpu.VMEM((m, d), bf16),
            pltpu.VMEM((N_STREAMS, N_DEV - 1, qh, d), bf16),
            pltpu.VMEM((N_STREAMS, N_DEV, qh, d), bf16),
            pltpu.VMEM((m, n), bf16),
            pltpu.SemaphoreType.DMA((N_DEV,)),
            pltpu.SemaphoreType.DMA((N_STREAMS, N_DEV - 1, 2)),
            pltpu.SemaphoreType.DMA((N_STREAMS, N_DEV - 1, 2)),
            pltpu.SemaphoreType.DMA((N_STREAMS, N_DEV - 1)),
            pltpu.SemaphoreType.DMA((N_STREAMS, N_DEV - 1)),
        ],
        compiler_params=pltpu.CompilerParams(collective_id=0),
    )
    return call(x.astype(bf16), W1.astype(bf16), W2.astype(bf16))


# device time: 28931 ns/iter; 1.0038x vs baseline; 1.0038x over previous
import jax
import jax.numpy as jnp
from jax import lax
from jax.experimental import pallas as pl
from jax.experimental.pallas import tpu as pltpu

N_DEV = 4
N_STREAMS = 4
ORDER = (0, 2, 1, 3)


def kernel(x, W1, W2):
    m, _ = x.shape
    d = W1.shape[1]
    n = W2.shape[1]
    mc = m // N_DEV
    qh = mc // N_STREAMS
    bf16 = jnp.bfloat16
    f32 = jnp.float32

    def body(x_ref, w1_ref, w2_ref, out_ref,
             h_ref, comm_ref, ag_ref,
             rs_send, rs_recv, ag_send, ag_recv):
        w1 = w1_ref[...]
        w2 = w2_ref[...]
        p = lax.axis_index("i")
        left = lax.rem(p + N_DEV - 1, N_DEV)
        right = lax.rem(p + 1, N_DEV)

        def mod4(v):
            return lax.rem(v + 4 * N_DEV, N_DEV)

        def is_r(st):
            return st < 2

        def row_start(c, st):
            return c * mc + st * qh

        def h_q(c, st):
            return h_ref[pl.ds(row_start(c, st), qh), :]

        def nbr(st):
            return right if is_r(st) else left

        def rs_id(st, s):
            return mod4(p - s - 1) if is_r(st) else mod4(p + s + 1)

        def ag_id(st, t):
            return mod4(p - t) if is_r(st) else mod4(p + t)

        def make(src_ref, buf, st, slot_dst, send_sems, recv_sems, hop):
            return pltpu.make_async_remote_copy(
                src_ref=src_ref,
                dst_ref=buf.at[st, slot_dst],
                send_sem=send_sems.at[st, hop],
                recv_sem=recv_sems.at[st, hop],
                device_id=(nbr(st),),
                device_id_type=pl.DeviceIdType.MESH,
            )

        sh = qh // 2

        def make_rs(src_ref, st, slot_dst, hop, sub):
            return pltpu.make_async_remote_copy(
                src_ref=src_ref,
                dst_ref=comm_ref.at[st, slot_dst, pl.ds(sub * sh, sh)],
                send_sem=rs_send.at[st, hop, sub],
                recv_sem=rs_recv.at[st, hop, sub],
                device_id=(nbr(st),),
                device_id_type=pl.DeviceIdType.MESH,
            )

        barrier = pltpu.get_barrier_semaphore()
        for b in (left, right):
            pl.semaphore_signal(barrier, inc=1, device_id=(b,),
                                device_id_type=pl.DeviceIdType.MESH)

        def gemm1_q(st):
            start = row_start(p, st)
            h_ref[pl.ds(start, qh), :] = jnp.dot(
                x_ref[pl.ds(start, qh), :], w1,
                preferred_element_type=f32).astype(bf16)

        def gemm1(c):
            start = c * mc
            h_ref[pl.ds(start, mc), :] = jnp.dot(
                x_ref[pl.ds(start, mc), :], w1,
                preferred_element_type=f32).astype(bf16)

        rs_desc = [[[None, None] for _ in range(N_DEV - 1)]
                   for _ in range(N_STREAMS)]
        ag_desc = [[None] * (N_DEV - 1) for _ in range(N_STREAMS)]

        def start_rs0(st):
            for sub in (0, 1):
                rs_desc[st][0][sub] = make_rs(
                    h_ref.at[pl.ds(row_start(p, st) + sub * sh, sh)],
                    st, 0, 0, sub)
                rs_desc[st][0][sub].start()

        gemm1_q(0)
        gemm1_q(2)
        pl.semaphore_wait(barrier, 2)
        start_rs0(0)
        start_rs0(2)
        gemm1_q(1)
        gemm1_q(3)
        start_rs0(1)
        start_rs0(3)

        gemm1(mod4(p + 3))
        gemm1(mod4(p + 1))

        RS_EVENTS = ((0, 0), (2, 0), (0, 1), (2, 1),
                     (1, 0), (3, 0), (1, 1), (3, 1))
        for s in range(N_DEV - 1):
            for st, sub in RS_EVENTS:
                rs_desc[st][s][sub].wait_recv()
                rows = pl.ds(sub * sh, sh)
                c = rs_id(st, s)
                acc = (comm_ref[st, s, rows, :]
                       + h_ref[pl.ds(row_start(c, st) + sub * sh, sh), :])
                if s < N_DEV - 2:
                    comm_ref[st, s, rows, :] = acc
                    rs_desc[st][s + 1][sub] = make_rs(
                        comm_ref.at[st, s, rows], st, s + 1, s + 1, sub)
                    rs_desc[st][s + 1][sub].start()
                else:
                    ag_ref[st, 3, rows, :] = acc
                    if sub == 1:
                        ag_desc[st][0] = make(
                            ag_ref.at[st, 3], ag_ref, st, 0,
                            ag_send, ag_recv, 0)
                        ag_desc[st][0].start()
            if s == 0:
                gemm1(mod4(p + 2))

        def gemm2(src_val, c, st):
            out_ref[pl.ds(row_start(c, st), qh), :] = jnp.dot(
                src_val, w2, preferred_element_type=f32
            ).astype(bf16)

        for st in ORDER:
            own_c = mod4(p + 1) if is_r(st) else mod4(p - 1)
            gemm2(ag_ref[st, 3], own_c, st)

        for t in range(N_DEV - 1):
            if t < N_DEV - 2:
                for st in ORDER:
                    ag_desc[st][t].wait_recv()
                    ag_desc[st][t + 1] = make(
                        ag_ref.at[st, t], ag_ref, st, t + 1,
                        ag_send, ag_recv, t + 1)
                    ag_desc[st][t + 1].start()
                for st in ORDER:
                    gemm2(ag_ref[st, t], ag_id(st, t), st)
            else:
                for st in ORDER:
                    ag_desc[st][t].wait_recv()
                    gemm2(ag_ref[st, t], ag_id(st, t), st)

        for st in range(N_STREAMS):
            for s in range(N_DEV - 1):
                rs_desc[st][s][0].wait_send()
                rs_desc[st][s][1].wait_send()
                ag_desc[st][s].wait_send()

    call = pl.pallas_call(
        body,
        out_shape=jax.ShapeDtypeStruct((m, n), bf16),
        in_specs=[pl.BlockSpec(memory_space=pltpu.VMEM)] * 3,
        out_specs=pl.BlockSpec(memory_space=pltpu.VMEM),
        scratch_shapes=[
            pltpu.VMEM((m, d), bf16),
            pltpu.VMEM((N_STREAMS, N_DEV - 1, qh, d), bf16),
            pltpu.VMEM((N_STREAMS, N_DEV, qh, d), bf16),
            pltpu.SemaphoreType.DMA((N_STREAMS, N_DEV - 1, 2)),
            pltpu.SemaphoreType.DMA((N_STREAMS, N_DEV - 1, 2)),
            pltpu.SemaphoreType.DMA((N_STREAMS, N_DEV - 1)),
            pltpu.SemaphoreType.DMA((N_STREAMS, N_DEV - 1)),
        ],
        compiler_params=pltpu.CompilerParams(collective_id=0),
    )
    return call(x.astype(bf16), W1.astype(bf16), W2.astype(bf16))


# device time: 28565 ns/iter; 1.0167x vs baseline; 1.0128x over previous
import jax
import jax.numpy as jnp
from jax import lax
from jax.experimental import pallas as pl
from jax.experimental.pallas import tpu as pltpu

N_DEV = 4
N_STREAMS = 4
ORDER = (0, 2, 1, 3)


def kernel(x, W1, W2):
    m, _ = x.shape
    d = W1.shape[1]
    n = W2.shape[1]
    mc = m // N_DEV
    qh = mc // N_STREAMS
    bf16 = jnp.bfloat16
    f32 = jnp.float32

    def body(x_ref, w1_ref, w2_ref, out_ref,
             h_ref, comm_ref, ag_ref,
             rs_send, rs_recv, ag_send, ag_recv):
        w1 = w1_ref[...]
        w2 = w2_ref[...]
        p = lax.axis_index("i")
        left = lax.rem(p + N_DEV - 1, N_DEV)
        right = lax.rem(p + 1, N_DEV)

        def mod4(v):
            return lax.rem(v + 4 * N_DEV, N_DEV)

        def is_r(st):
            return st < 2

        def row_start(c, st):
            return c * mc + st * qh

        def h_q(c, st):
            return h_ref[pl.ds(row_start(c, st), qh), :]

        def nbr(st):
            return right if is_r(st) else left

        def rs_id(st, s):
            return mod4(p - s - 1) if is_r(st) else mod4(p + s + 1)

        def ag_id(st, t):
            return mod4(p - t) if is_r(st) else mod4(p + t)

        def make(src_ref, buf, st, slot_dst, send_sems, recv_sems, hop):
            return pltpu.make_async_remote_copy(
                src_ref=src_ref,
                dst_ref=buf.at[st, slot_dst],
                send_sem=send_sems.at[st, hop],
                recv_sem=recv_sems.at[st, hop],
                device_id=(nbr(st),),
                device_id_type=pl.DeviceIdType.MESH,
            )

        sh = qh // 2

        def make_rs(src_ref, st, slot_dst, hop, sub):
            return pltpu.make_async_remote_copy(
                src_ref=src_ref,
                dst_ref=comm_ref.at[st, slot_dst, pl.ds(sub * sh, sh)],
                send_sem=rs_send.at[st, hop, sub],
                recv_sem=rs_recv.at[st, hop, sub],
                device_id=(nbr(st),),
                device_id_type=pl.DeviceIdType.MESH,
            )

        def make_ag(st, slot_src, slot_dst, hop, sub):
            rows = pl.ds(sub * sh, sh)
            return pltpu.make_async_remote_copy(
                src_ref=ag_ref.at[st, slot_src, rows],
                dst_ref=ag_ref.at[st, slot_dst, rows],
                send_sem=ag_send.at[st, hop, sub],
                recv_sem=ag_recv.at[st, hop, sub],
                device_id=(nbr(st),),
                device_id_type=pl.DeviceIdType.MESH,
            )

        barrier = pltpu.get_barrier_semaphore()
        for b in (left, right):
            pl.semaphore_signal(barrier, inc=1, device_id=(b,),
                                device_id_type=pl.DeviceIdType.MESH)

        def gemm1_q(st):
            start = row_start(p, st)
            h_ref[pl.ds(start, qh), :] = jnp.dot(
                x_ref[pl.ds(start, qh), :], w1,
                preferred_element_type=f32).astype(bf16)

        def gemm1(c):
            start = c * mc
            h_ref[pl.ds(start, mc), :] = jnp.dot(
                x_ref[pl.ds(start, mc), :], w1,
                preferred_element_type=f32).astype(bf16)

        rs_desc = [[[None, None] for _ in range(N_DEV - 1)]
                   for _ in range(N_STREAMS)]
        ag_desc = [[[None, None] for _ in range(N_DEV - 1)]
                   for _ in range(N_STREAMS)]

        def start_rs0(st):
            for sub in (0, 1):
                rs_desc[st][0][sub] = make_rs(
                    h_ref.at[pl.ds(row_start(p, st) + sub * sh, sh)],
                    st, 0, 0, sub)
                rs_desc[st][0][sub].start()

        gemm1_q(0)
        gemm1_q(2)
        pl.semaphore_wait(barrier, 2)
        start_rs0(0)
        start_rs0(2)
        gemm1_q(1)
        gemm1_q(3)
        start_rs0(1)
        start_rs0(3)

        gemm1(mod4(p + 3))
        gemm1(mod4(p + 1))

        RS_EVENTS = ((0, 0), (2, 0), (0, 1), (2, 1),
                     (1, 0), (3, 0), (1, 1), (3, 1))
        for s in range(N_DEV - 1):
            for st, sub in RS_EVENTS:
                rs_desc[st][s][sub].wait_recv()
                rows = pl.ds(sub * sh, sh)
                c = rs_id(st, s)
                acc = (comm_ref[st, s, rows, :]
                       + h_ref[pl.ds(row_start(c, st) + sub * sh, sh), :])
                if s < N_DEV - 2:
                    comm_ref[st, s, rows, :] = acc
                    rs_desc[st][s + 1][sub] = make_rs(
                        comm_ref.at[st, s, rows], st, s + 1, s + 1, sub)
                    rs_desc[st][s + 1][sub].start()
                else:
                    ag_ref[st, 3, rows, :] = acc
                    ag_desc[st][0][sub] = make_ag(st, 3, 0, 0, sub)
                    ag_desc[st][0][sub].start()
            if s == 0:
                gemm1(mod4(p + 2))

        def gemm2(src_val, c, st):
            out_ref[pl.ds(row_start(c, st), qh), :] = jnp.dot(
                src_val, w2, preferred_element_type=f32
            ).astype(bf16)

        for st in ORDER:
            own_c = mod4(p + 1) if is_r(st) else mod4(p - 1)
            gemm2(ag_ref[st, 3], own_c, st)

        for t in range(N_DEV - 1):
            if t < N_DEV - 2:
                for st, sub in RS_EVENTS:
                    ag_desc[st][t][sub].wait_recv()
                    ag_desc[st][t + 1][sub] = make_ag(st, t, t + 1,
                                                      t + 1, sub)
                    ag_desc[st][t + 1][sub].start()
                for st in ORDER:
                    gemm2(ag_ref[st, t], ag_id(st, t), st)
            else:
                for st in ORDER:
                    ag_desc[st][t][0].wait_recv()
                    ag_desc[st][t][1].wait_recv()
                    gemm2(ag_ref[st, t], ag_id(st, t), st)

        for st in range(N_STREAMS):
            for s in range(N_DEV - 1):
                for sub in (0, 1):
                    rs_desc[st][s][sub].wait_send()
                    ag_desc[st][s][sub].wait_send()

    call = pl.pallas_call(
        body,
        out_shape=jax.ShapeDtypeStruct((m, n), bf16),
        in_specs=[pl.BlockSpec(memory_space=pltpu.VMEM)] * 3,
        out_specs=pl.BlockSpec(memory_space=pltpu.VMEM),
        scratch_shapes=[
            pltpu.VMEM((m, d), bf16),
            pltpu.VMEM((N_STREAMS, N_DEV - 1, qh, d), bf16),
            pltpu.VMEM((N_STREAMS, N_DEV, qh, d), bf16),
            pltpu.SemaphoreType.DMA((N_STREAMS, N_DEV - 1, 2)),
            pltpu.SemaphoreType.DMA((N_STREAMS, N_DEV - 1, 2)),
            pltpu.SemaphoreType.DMA((N_STREAMS, N_DEV - 1, 2)),
            pltpu.SemaphoreType.DMA((N_STREAMS, N_DEV - 1, 2)),
        ],
        compiler_params=pltpu.CompilerParams(collective_id=0),
    )
    return call(x.astype(bf16), W1.astype(bf16), W2.astype(bf16))
